# Initial kernel scaffold; baseline (speedup 1.0000x reference)
import functools

import jax
import jax.numpy as jnp
from jax import lax
from jax.experimental import pallas as pl
from jax.experimental.pallas import tpu as pltpu
from jax.experimental.pallas import tpu_sc as plsc

M = 10000
D = 128
E = 160000
MP = 10240
EP = 163840
EPT = EP // 32
NCH = EPT // 128
RPT = MP // 16
BQ = 256
BK = 256
NBQ = MP // BQ
NBK = MP // BK
KEY_SHIFT = 14


def _leaky(x):
    return jnp.maximum(x, 0.0) + 0.01 * jnp.minimum(x, 0.0)


def _proj_body(z_b, wgT, wqT, wkT, wcT, a1, a2, wg_o, q_o, k_o, c_o, u_o, v_o):
    zb = z_b[...]
    wg = jnp.dot(zb, wgT[...], preferred_element_type=jnp.float32)
    wg_o[...] = wg
    q_o[...] = jnp.dot(zb, wqT[...], preferred_element_type=jnp.float32)
    k_o[...] = jnp.dot(zb, wkT[...], preferred_element_type=jnp.float32)
    c_o[...] = jnp.dot(zb, wcT[...], preferred_element_type=jnp.float32)
    u_o[...] = jnp.dot(wg, a1[...], preferred_element_type=jnp.float32)
    v_o[...] = jnp.dot(wg, a2[...], preferred_element_type=jnp.float32)


def _projections(zp, wgT, wqT, wkT, wcT, a1, a2):
    full = pl.BlockSpec((D, D), lambda i: (0, 0))
    colv = pl.BlockSpec((D, 1), lambda i: (0, 0))
    row = pl.BlockSpec((BQ, D), lambda i: (i, 0))
    rowv = pl.BlockSpec((BQ, 1), lambda i: (i, 0))
    mat = jax.ShapeDtypeStruct((MP, D), jnp.float32)
    vec = jax.ShapeDtypeStruct((MP, 1), jnp.float32)
    return pl.pallas_call(
        _proj_body,
        grid=(NBQ,),
        in_specs=[row, full, full, full, full, colv, colv],
        out_specs=[row, row, row, row, rowv, rowv],
        out_shape=[mat, mat, mat, mat, vec, vec],
    )(zp, wgT, wqT, wkT, wcT, a1, a2)


def _sc_body(wg_hbm, u_hbm, v_hbm, s1d_hbm, d1d_hbm, s2d_hbm, dl2d_hbm,
             zmat_hbm, zvec_hbm,
             local_out, denom_out,
             u_v, v_v, es_v, s1v, d1v, s2v, dl2v, rows_v,
             local_sp, denom_sp):
    c = lax.axis_index("c")
    s = lax.axis_index("s")
    w = s * 2 + c

    pltpu.sync_copy(zmat_hbm.at[pl.ds(s * RPT, RPT)],
                    local_sp.at[pl.ds(s * RPT, RPT)])
    pltpu.sync_copy(zvec_hbm.at[pl.ds(s * RPT, RPT)],
                    denom_sp.at[pl.ds(s * RPT, RPT)])

    pltpu.sync_copy(u_hbm, u_v)
    pltpu.sync_copy(v_hbm, v_v)
    pltpu.sync_copy(s1d_hbm.at[pl.ds(w * EPT, EPT)], s1v)
    pltpu.sync_copy(d1d_hbm.at[pl.ds(w * EPT, EPT)], d1v)
    pltpu.sync_copy(s2d_hbm.at[pl.ds(w * NCH, NCH)], s2v)
    pltpu.sync_copy(dl2d_hbm.at[pl.ds(w * NCH, NCH)], dl2v)

    def vmax_step(i, acc):
        return jnp.maximum(acc, v_v[pl.ds(i * 16, 16)])
    vmax16 = lax.fori_loop(0, MP // 16, vmax_step,
                           jnp.full((16,), -1e30, jnp.float32))
    vmaxb = jnp.full((16,), jnp.max(vmax16))

    def p1(i, carry):
        si = s1v[pl.ds(i * 16, 16)]
        di = d1v[pl.ds(i * 16, 16)]
        us = plsc.load_gather(u_v, [si])
        vd = plsc.load_gather(v_v, [di])
        e = _leaky(us + vd)
        shift = _leaky(us + vmaxb)
        es_v[pl.ds(i * 16, 16)] = jnp.exp(e - shift)
        return carry
    lax.fori_loop(0, EPT // 16, p1, 0)

    plsc.subcore_barrier()

    def p1s(j, carry):
        pltpu.sync_copy(es_v.at[pl.ds(j * 128, 128)],
                        denom_sp.at[s2v.at[j]], add=True)
        return carry
    lax.fori_loop(0, NCH, p1s, 0)

    def p2(j, carry):
        pltpu.sync_copy(wg_hbm.at[dl2v.at[j]], rows_v)

        def scale(r, cc):
            idx = jnp.full((16,), j * 128 + r, jnp.int32)
            esb = plsc.load_gather(es_v, [idx])
            for g in range(8):
                rows_v[r, pl.ds(g * 16, 16)] = (
                    rows_v[r, pl.ds(g * 16, 16)] * esb)
            return cc
        lax.fori_loop(0, 128, scale, 0)

        pltpu.sync_copy(rows_v, local_sp.at[s2v.at[j]], add=True)
        return carry
    lax.fori_loop(0, NCH, p2, 0)

    plsc.subcore_barrier()
    pltpu.sync_copy(local_sp.at[pl.ds(s * RPT, RPT)],
                    local_out.at[c].at[pl.ds(s * RPT, RPT)])
    pltpu.sync_copy(denom_sp.at[pl.ds(s * RPT, RPT)],
                    denom_out.at[c].at[pl.ds(s * RPT, RPT)])


def _sc_local(wg, u1, v1, s1d, d1d, s2d, dl2d, zmat, zvec):
    mesh = plsc.VectorSubcoreMesh(core_axis_name="c", subcore_axis_name="s",
                                  num_cores=2, num_subcores=16)
    return pl.kernel(
        _sc_body,
        out_type=(jax.ShapeDtypeStruct((2, MP, D), jnp.float32),
                  jax.ShapeDtypeStruct((2, MP), jnp.float32)),
        mesh=mesh,
        scratch_types=[
            pltpu.VMEM((MP,), jnp.float32),
            pltpu.VMEM((MP,), jnp.float32),
            pltpu.VMEM((EPT,), jnp.float32),
            pltpu.VMEM((EPT,), jnp.int32),
            pltpu.VMEM((EPT,), jnp.int32),
            pltpu.VMEM((NCH, 128), jnp.int32),
            pltpu.VMEM((NCH, 128), jnp.int32),
            pltpu.VMEM((128, D), jnp.float32),
            pltpu.VMEM_SHARED((MP, D), jnp.float32),
            pltpu.VMEM_SHARED((MP,), jnp.float32),
        ],
    )(wg, u1, v1, s1d, d1d, s2d, dl2d, zmat, zvec)


def _flash_body(q_b, k_b, c_b, z_b, l0_b, l1_b, dn_b, o_b, acc, mv, lv):
    j = pl.program_id(1)

    @pl.when(j == 0)
    def _():
        acc[...] = jnp.zeros_like(acc)
        mv[...] = jnp.full_like(mv, -1e30)
        lv[...] = jnp.zeros_like(lv)

    scale = 1.0 / (float(D) ** 0.5)
    s = lax.dot_general(q_b[...], k_b[...], (((1,), (1,)), ((), ())),
                        preferred_element_type=jnp.float32) * scale
    col = lax.broadcasted_iota(jnp.int32, (BQ, BK), 1) + j * BK
    s = jnp.where(col < M, s, -1e30)
    m_prev = mv[...]
    m_new = jnp.maximum(m_prev, jnp.max(s, axis=1, keepdims=True))
    corr = jnp.exp(m_prev - m_new)
    p = jnp.exp(s - m_new)
    lv[...] = lv[...] * corr + jnp.sum(p, axis=1, keepdims=True)
    acc[...] = acc[...] * corr + jnp.dot(p, c_b[...],
                                         preferred_element_type=jnp.float32)
    mv[...] = m_new

    @pl.when(j == NBK - 1)
    def _():
        glob = acc[...] / lv[...]
        dn = dn_b[...]
        dn = jnp.where(dn == 0.0, 1.0, dn)
        local = (l0_b[...] + l1_b[...]) / dn
        o_b[...] = _leaky(local + glob + z_b[...])


def _flash(q, k, c, zp, l0, l1, dn):
    rowi = pl.BlockSpec((BQ, D), lambda i, j: (i, 0))
    rowj = pl.BlockSpec((BK, D), lambda i, j: (j, 0))
    veci = pl.BlockSpec((BQ, 1), lambda i, j: (i, 0))
    return pl.pallas_call(
        _flash_body,
        grid=(NBQ, NBK),
        in_specs=[rowi, rowj, rowj, rowi, rowi, rowi, veci],
        out_specs=rowi,
        out_shape=jax.ShapeDtypeStruct((MP, D), jnp.float32),
        scratch_shapes=[
            pltpu.VMEM((BQ, D), jnp.float32),
            pltpu.VMEM((BQ, 1), jnp.float32),
            pltpu.VMEM((BQ, 1), jnp.float32),
        ],
        compiler_params=pltpu.CompilerParams(
            dimension_semantics=("parallel", "arbitrary")),
    )(q, k, c, zp, l0, l1, dn)


def kernel(z, edge_index, Wg, a, Wc, Wq, Wk):
    m, d = z.shape
    assert (m, d, edge_index.shape[1]) == (M, D, E)

    zp = jnp.pad(z, ((0, MP - m), (0, 0)))
    a1 = a[:d].reshape(d, 1).astype(jnp.float32)
    a2 = a[d:].reshape(d, 1).astype(jnp.float32)
    wg, q, k, c, u2, v2 = _projections(
        zp, Wg.T, Wq.T, Wk.T, Wc.T, a1, a2)
    u1 = u2.reshape(MP)
    v1 = v2.reshape(MP)

    src = edge_index[0].astype(jnp.int32)
    dst = edge_index[1].astype(jnp.int32)
    sk = jnp.sort((src << KEY_SHIFT) | dst)
    ssrc = sk >> KEY_SHIFT
    sdst = sk & ((1 << KEY_SHIFT) - 1)
    first = jnp.concatenate(
        [jnp.ones((1,), bool), sk[1:] != sk[:-1]])
    sdstl = jnp.where(first, sdst, M)
    padv = jnp.full((EP - E,), M, jnp.int32)
    ssrc = jnp.concatenate([ssrc, padv])
    sdst = jnp.concatenate([sdst, padv])
    sdstl = jnp.concatenate([sdstl, padv])

    zmat = jnp.zeros((MP, D), jnp.float32)
    zvec = jnp.zeros((MP,), jnp.float32)
    localp, denomp = _sc_local(
        wg, u1, v1, ssrc, sdst,
        ssrc.reshape(EP // 128, 128), sdstl.reshape(EP // 128, 128),
        zmat, zvec)

    dn = (denomp[0] + denomp[1]).reshape(MP, 1)
    out = _flash(q, k, c, zp, localp[0], localp[1], dn)
    return out[:m]


# speedup vs baseline: 2.1744x; 2.1744x over previous
import functools

import jax
import jax.numpy as jnp
from jax import lax
from jax.experimental import pallas as pl
from jax.experimental.pallas import tpu as pltpu
from jax.experimental.pallas import tpu_sc as plsc

M = 10000
D = 128
E = 160000
MP = 10240
EP = 163840
EPT = EP // 32
NCH = EPT // 128
RPT = MP // 16
BQ = 256
BK = 256
NBQ = MP // BQ
NBK = MP // BK
KEY_SHIFT = 14


def _leaky(x):
    return jnp.maximum(x, 0.0) + 0.01 * jnp.minimum(x, 0.0)


def _proj_body(z_b, wgT, wqT, wkT, wcT, a1, a2, wg_o, q_o, k_o, c_o, u_o, v_o):
    zb = z_b[...]
    wg = jnp.dot(zb, wgT[...], preferred_element_type=jnp.float32)
    wg_o[...] = wg
    q_o[...] = jnp.dot(zb, wqT[...], preferred_element_type=jnp.float32)
    k_o[...] = jnp.dot(zb, wkT[...], preferred_element_type=jnp.float32)
    c_o[...] = jnp.dot(zb, wcT[...], preferred_element_type=jnp.float32)
    u_o[...] = jnp.dot(wg, a1[...], preferred_element_type=jnp.float32)
    v_o[...] = jnp.dot(wg, a2[...], preferred_element_type=jnp.float32)


def _projections(zp, wgT, wqT, wkT, wcT, a1, a2):
    full = pl.BlockSpec((D, D), lambda i: (0, 0))
    colv = pl.BlockSpec((D, 1), lambda i: (0, 0))
    row = pl.BlockSpec((BQ, D), lambda i: (i, 0))
    rowv = pl.BlockSpec((BQ, 1), lambda i: (i, 0))
    mat = jax.ShapeDtypeStruct((MP, D), jnp.float32)
    vec = jax.ShapeDtypeStruct((MP, 1), jnp.float32)
    return pl.pallas_call(
        _proj_body,
        grid=(NBQ,),
        in_specs=[row, full, full, full, full, colv, colv],
        out_specs=[row, row, row, row, rowv, rowv],
        out_shape=[mat, mat, mat, mat, vec, vec],
    )(zp, wgT, wqT, wkT, wcT, a1, a2)


def _sc_body(wg_hbm, u_hbm, v_hbm, vmax_hbm, s2d_hbm, d2d_hbm, dl2d_hbm,
             zmat_hbm, zvec_hbm,
             local_out, denom_out,
             vm_v, es_v, s2v, d2v, dl2v, uvals_v, vvals_v, rows_v,
             u_sp, v_sp, local_sp, denom_sp):
    c = lax.axis_index("c")
    s = lax.axis_index("s")
    w = s * 2 + c

    pltpu.sync_copy(zmat_hbm.at[pl.ds(s * RPT, RPT)],
                    local_sp.at[pl.ds(s * RPT, RPT)])
    pltpu.sync_copy(zvec_hbm.at[pl.ds(s * RPT, RPT)],
                    denom_sp.at[pl.ds(s * RPT, RPT)])
    pltpu.sync_copy(u_hbm.at[pl.ds(s * RPT, RPT)],
                    u_sp.at[pl.ds(s * RPT, RPT)])
    pltpu.sync_copy(v_hbm.at[pl.ds(s * RPT, RPT)],
                    v_sp.at[pl.ds(s * RPT, RPT)])

    pltpu.sync_copy(vmax_hbm, vm_v)
    pltpu.sync_copy(s2d_hbm.at[pl.ds(w * NCH, NCH)], s2v)
    pltpu.sync_copy(d2d_hbm.at[pl.ds(w * NCH, NCH)], d2v)
    pltpu.sync_copy(dl2d_hbm.at[pl.ds(w * NCH, NCH)], dl2v)

    vmaxb = vm_v[...]

    plsc.subcore_barrier()

    def p1(j, carry):
        pltpu.sync_copy(u_sp.at[s2v.at[j]], uvals_v)
        pltpu.sync_copy(v_sp.at[d2v.at[j]], vvals_v)
        for g in range(8):
            us = uvals_v[pl.ds(g * 16, 16)]
            vd = vvals_v[pl.ds(g * 16, 16)]
            e = _leaky(us + vd)
            shift = _leaky(us + vmaxb)
            es_v[pl.ds(j * 128 + g * 16, 16)] = jnp.exp(e - shift)
        pltpu.sync_copy(es_v.at[pl.ds(j * 128, 128)],
                        denom_sp.at[s2v.at[j]], add=True)
        return carry
    lax.fori_loop(0, NCH, p1, 0)

    def p2(j, carry):
        pltpu.sync_copy(wg_hbm.at[dl2v.at[j]], rows_v)

        def scale(r, cc):
            idx = jnp.full((16,), j * 128 + r, jnp.int32)
            esb = plsc.load_gather(es_v, [idx])
            for g in range(D // 16):
                rows_v[r, pl.ds(g * 16, 16)] = (
                    rows_v[r, pl.ds(g * 16, 16)] * esb)
            return cc
        lax.fori_loop(0, 128, scale, 0)

        pltpu.sync_copy(rows_v, local_sp.at[s2v.at[j]], add=True)
        return carry
    lax.fori_loop(0, NCH, p2, 0)

    plsc.subcore_barrier()
    pltpu.sync_copy(local_sp.at[pl.ds(s * RPT, RPT)],
                    local_out.at[c].at[pl.ds(s * RPT, RPT)])
    pltpu.sync_copy(denom_sp.at[pl.ds(s * RPT, RPT)],
                    denom_out.at[c].at[pl.ds(s * RPT, RPT)])


def _sc_local(wg, u1, v1, vmax16, s2d, d2d, dl2d, zmat, zvec):
    mesh = plsc.VectorSubcoreMesh(core_axis_name="c", subcore_axis_name="s",
                                  num_cores=2, num_subcores=16)
    return pl.kernel(
        _sc_body,
        out_type=(jax.ShapeDtypeStruct((2, MP, D), jnp.float32),
                  jax.ShapeDtypeStruct((2, MP), jnp.float32)),
        mesh=mesh,
        scratch_types=[
            pltpu.VMEM((16,), jnp.float32),
            pltpu.VMEM((EPT,), jnp.float32),
            pltpu.VMEM((NCH, 128), jnp.int32),
            pltpu.VMEM((NCH, 128), jnp.int32),
            pltpu.VMEM((NCH, 128), jnp.int32),
            pltpu.VMEM((128,), jnp.float32),
            pltpu.VMEM((128,), jnp.float32),
            pltpu.VMEM((128, D), jnp.float32),
            pltpu.VMEM_SHARED((MP,), jnp.float32),
            pltpu.VMEM_SHARED((MP,), jnp.float32),
            pltpu.VMEM_SHARED((MP, D), jnp.float32),
            pltpu.VMEM_SHARED((MP,), jnp.float32),
        ],
        compiler_params=pltpu.CompilerParams(needs_layout_passes=False),
    )(wg, u1, v1, vmax16, s2d, d2d, dl2d, zmat, zvec)


def _flash_body(q_b, k_b, c_b, z_b, l0_b, l1_b, dn_b, o_b, acc, mv, lv):
    j = pl.program_id(1)

    @pl.when(j == 0)
    def _():
        acc[...] = jnp.zeros_like(acc)
        mv[...] = jnp.full_like(mv, -1e30)
        lv[...] = jnp.zeros_like(lv)

    scale = 1.0 / (float(D) ** 0.5)
    s = lax.dot_general(q_b[...], k_b[...], (((1,), (1,)), ((), ())),
                        preferred_element_type=jnp.float32) * scale
    col = lax.broadcasted_iota(jnp.int32, (BQ, BK), 1) + j * BK
    s = jnp.where(col < M, s, -1e30)
    m_prev = mv[...]
    m_new = jnp.maximum(m_prev, jnp.max(s, axis=1, keepdims=True))
    corr = jnp.exp(m_prev - m_new)
    p = jnp.exp(s - m_new)
    lv[...] = lv[...] * corr + jnp.sum(p, axis=1, keepdims=True)
    acc[...] = acc[...] * corr + jnp.dot(p, c_b[...],
                                         preferred_element_type=jnp.float32)
    mv[...] = m_new

    @pl.when(j == NBK - 1)
    def _():
        glob = acc[...] / lv[...]
        dn = dn_b[...]
        dn = jnp.where(dn == 0.0, 1.0, dn)
        local = (l0_b[...] + l1_b[...]) / dn
        o_b[...] = _leaky(local + glob + z_b[...])


def _flash(q, k, c, zp, l0, l1, dn):
    rowi = pl.BlockSpec((BQ, D), lambda i, j: (i, 0))
    rowj = pl.BlockSpec((BK, D), lambda i, j: (j, 0))
    veci = pl.BlockSpec((BQ, 1), lambda i, j: (i, 0))
    return pl.pallas_call(
        _flash_body,
        grid=(NBQ, NBK),
        in_specs=[rowi, rowj, rowj, rowi, rowi, rowi, veci],
        out_specs=rowi,
        out_shape=jax.ShapeDtypeStruct((MP, D), jnp.float32),
        scratch_shapes=[
            pltpu.VMEM((BQ, D), jnp.float32),
            pltpu.VMEM((BQ, 1), jnp.float32),
            pltpu.VMEM((BQ, 1), jnp.float32),
        ],
        compiler_params=pltpu.CompilerParams(
            dimension_semantics=("parallel", "arbitrary")),
    )(q, k, c, zp, l0, l1, dn)


def kernel(z, edge_index, Wg, a, Wc, Wq, Wk):
    m, d = z.shape
    assert (m, d, edge_index.shape[1]) == (M, D, E)

    zp = jnp.pad(z, ((0, MP - m), (0, 0)))
    a1 = a[:d].reshape(d, 1).astype(jnp.float32)
    a2 = a[d:].reshape(d, 1).astype(jnp.float32)
    wg, q, k, c, u2, v2 = _projections(
        zp, Wg.T, Wq.T, Wk.T, Wc.T, a1, a2)
    u1 = u2.reshape(MP)
    v1 = v2.reshape(MP)
    vmax16 = jnp.full((16,), jnp.max(v2), jnp.float32)

    src = edge_index[0].astype(jnp.int32)
    dst = edge_index[1].astype(jnp.int32)
    sk = jnp.sort((src << KEY_SHIFT) | dst)
    ssrc = sk >> KEY_SHIFT
    sdst = sk & ((1 << KEY_SHIFT) - 1)
    first = jnp.concatenate(
        [jnp.ones((1,), bool), sk[1:] != sk[:-1]])
    sdstl = jnp.where(first, sdst, M)
    padv = jnp.full((EP - E,), M, jnp.int32)
    ssrc = jnp.concatenate([ssrc, padv])
    sdst = jnp.concatenate([sdst, padv])
    sdstl = jnp.concatenate([sdstl, padv])

    zmat = jnp.zeros((MP, D), jnp.float32)
    zvec = jnp.zeros((MP,), jnp.float32)
    localp, denomp = _sc_local(
        wg, u1, v1, vmax16,
        ssrc.reshape(EP // 128, 128), sdst.reshape(EP // 128, 128),
        sdstl.reshape(EP // 128, 128), zmat, zvec)

    dn = (denomp[0] + denomp[1]).reshape(MP, 1)
    out = _flash(q, k, c, zp, localp[0], localp[1], dn)
    return out[:m]


# speedup vs baseline: 2.5834x; 1.1881x over previous
import functools

import jax
import jax.numpy as jnp
from jax import lax
from jax.experimental import pallas as pl
from jax.experimental.pallas import tpu as pltpu
from jax.experimental.pallas import tpu_sc as plsc

M = 10000
D = 128
E = 160000
MP = 10240
EP = 163840
EPT = EP // 32
NCH = EPT // 128
RPT = MP // 16
BQ = 256
BK = 256
NBQ = MP // BQ
NBK = MP // BK
KEY_SHIFT = 14


def _leaky(x):
    return jnp.maximum(x, 0.0) + 0.01 * jnp.minimum(x, 0.0)


def _proj_body(z_b, wgT, wqT, wkT, wcT, a1, a2, wg_o, q_o, k_o, c_o, u_o, v_o):
    zb = z_b[...]
    wg = jnp.dot(zb, wgT[...], preferred_element_type=jnp.float32)
    wg_o[...] = wg
    q_o[...] = jnp.dot(zb, wqT[...], preferred_element_type=jnp.float32)
    k_o[...] = jnp.dot(zb, wkT[...], preferred_element_type=jnp.float32)
    c_o[...] = jnp.dot(zb, wcT[...], preferred_element_type=jnp.float32)
    u_o[...] = jnp.dot(wg, a1[...], preferred_element_type=jnp.float32)
    v_o[...] = jnp.dot(wg, a2[...], preferred_element_type=jnp.float32)


def _projections(zp, wgT, wqT, wkT, wcT, a1, a2):
    full = pl.BlockSpec((D, D), lambda i: (0, 0))
    colv = pl.BlockSpec((D, 1), lambda i: (0, 0))
    row = pl.BlockSpec((BQ, D), lambda i: (i, 0))
    rowv = pl.BlockSpec((BQ, 1), lambda i: (i, 0))
    mat = jax.ShapeDtypeStruct((MP, D), jnp.float32)
    vec = jax.ShapeDtypeStruct((MP, 1), jnp.float32)
    return pl.pallas_call(
        _proj_body,
        grid=(NBQ,),
        in_specs=[row, full, full, full, full, colv, colv],
        out_specs=[row, row, row, row, rowv, rowv],
        out_shape=[mat, mat, mat, mat, vec, vec],
    )(zp, wgT, wqT, wkT, wcT, a1, a2)


def _sc_body(wg_hbm, u_hbm, v_hbm, vmax_hbm, s2d_hbm, d2d_hbm, dl2d_hbm,
             zmat_hbm, zvec_hbm,
             local_out, denom_out,
             vm_v, es_v, s2v, d2v, dl2v, uvals_v, vvals_v, rows_v,
             u_sp, v_sp, local_sp, denom_sp):
    c = lax.axis_index("c")
    s = lax.axis_index("s")
    w = s * 2 + c

    pltpu.sync_copy(zmat_hbm.at[pl.ds(s * RPT, RPT)],
                    local_sp.at[pl.ds(s * RPT, RPT)])
    pltpu.sync_copy(zvec_hbm.at[pl.ds(s * RPT, RPT)],
                    denom_sp.at[pl.ds(s * RPT, RPT)])
    pltpu.sync_copy(u_hbm.at[pl.ds(s * RPT, RPT)],
                    u_sp.at[pl.ds(s * RPT, RPT)])
    pltpu.sync_copy(v_hbm.at[pl.ds(s * RPT, RPT)],
                    v_sp.at[pl.ds(s * RPT, RPT)])

    pltpu.sync_copy(vmax_hbm, vm_v)
    pltpu.sync_copy(s2d_hbm.at[pl.ds(w * NCH, NCH)], s2v)
    pltpu.sync_copy(d2d_hbm.at[pl.ds(w * NCH, NCH)], d2v)
    pltpu.sync_copy(dl2d_hbm.at[pl.ds(w * NCH, NCH)], dl2v)

    vmaxb = vm_v[...]

    plsc.subcore_barrier()

    def p1(j, carry):
        pltpu.sync_copy(u_sp.at[s2v.at[j]], uvals_v)
        pltpu.sync_copy(v_sp.at[d2v.at[j]], vvals_v)
        for g in range(8):
            us = uvals_v[pl.ds(g * 16, 16)]
            vd = vvals_v[pl.ds(g * 16, 16)]
            e = _leaky(us + vd)
            shift = _leaky(us + vmaxb)
            es_v[pl.ds(j * 128 + g * 16, 16)] = jnp.exp(e - shift)
        pltpu.sync_copy(es_v.at[pl.ds(j * 128, 128)],
                        denom_sp.at[s2v.at[j]], add=True)
        return carry
    lax.fori_loop(0, NCH, p1, 0)

    def p2(j, carry):
        pltpu.sync_copy(wg_hbm.at[dl2v.at[j]], rows_v)

        def scale(r, cc):
            idx = jnp.full((16,), j * 128 + r, jnp.int32)
            esb = plsc.load_gather(es_v, [idx])
            for g in range(D // 16):
                rows_v[r, pl.ds(g * 16, 16)] = (
                    rows_v[r, pl.ds(g * 16, 16)] * esb)
            return cc
        lax.fori_loop(0, 128, scale, 0)

        pltpu.sync_copy(rows_v, local_sp.at[s2v.at[j]], add=True)
        return carry
    lax.fori_loop(0, NCH, p2, 0)

    plsc.subcore_barrier()
    pltpu.sync_copy(local_sp.at[pl.ds(s * RPT, RPT)],
                    local_out.at[c].at[pl.ds(s * RPT, RPT)])
    pltpu.sync_copy(denom_sp.at[pl.ds(s * RPT, RPT)],
                    denom_out.at[c].at[pl.ds(s * RPT, RPT)])


def _sc_local(wg, u1, v1, vmax16, s2d, d2d, dl2d, zmat, zvec):
    mesh = plsc.VectorSubcoreMesh(core_axis_name="c", subcore_axis_name="s",
                                  num_cores=2, num_subcores=16)
    return pl.kernel(
        _sc_body,
        out_type=(jax.ShapeDtypeStruct((2, MP, D), jnp.float32),
                  jax.ShapeDtypeStruct((2, MP), jnp.float32)),
        mesh=mesh,
        scratch_types=[
            pltpu.VMEM((16,), jnp.float32),
            pltpu.VMEM((EPT,), jnp.float32),
            pltpu.VMEM((NCH, 128), jnp.int32),
            pltpu.VMEM((NCH, 128), jnp.int32),
            pltpu.VMEM((NCH, 128), jnp.int32),
            pltpu.VMEM((128,), jnp.float32),
            pltpu.VMEM((128,), jnp.float32),
            pltpu.VMEM((128, D), jnp.float32),
            pltpu.VMEM_SHARED((MP,), jnp.float32),
            pltpu.VMEM_SHARED((MP,), jnp.float32),
            pltpu.VMEM_SHARED((MP, D), jnp.float32),
            pltpu.VMEM_SHARED((MP,), jnp.float32),
        ],
        compiler_params=pltpu.CompilerParams(needs_layout_passes=False),
    )(wg, u1, v1, vmax16, s2d, d2d, dl2d, zmat, zvec)


def _flash_body(q_b, k_b, c_b, o_b, acc, mv, lv):
    j = pl.program_id(1)

    @pl.when(j == 0)
    def _():
        acc[...] = jnp.zeros_like(acc)
        mv[...] = jnp.full_like(mv, -1e30)
        lv[...] = jnp.zeros_like(lv)

    scale = 1.0 / (float(D) ** 0.5)
    s = lax.dot_general(q_b[...], k_b[...], (((1,), (1,)), ((), ())),
                        preferred_element_type=jnp.float32) * scale
    col = lax.broadcasted_iota(jnp.int32, (BQ, BK), 1) + j * BK
    s = jnp.where(col < M, s, -1e30)
    m_prev = mv[...]
    m_new = jnp.maximum(m_prev, jnp.max(s, axis=1, keepdims=True))
    corr = jnp.exp(m_prev - m_new)
    p = jnp.exp(s - m_new).astype(jnp.bfloat16)
    lv[...] = lv[...] * corr + jnp.sum(p, axis=1, keepdims=True,
                                       dtype=jnp.float32)
    acc[...] = acc[...] * corr + jnp.dot(p, c_b[...],
                                         preferred_element_type=jnp.float32)

    mv[...] = m_new

    @pl.when(j == NBK - 1)
    def _():
        o_b[...] = acc[...] / lv[...]


def _flash(q, k, c):
    rowi = pl.BlockSpec((BQ, D), lambda i, j: (i, 0))
    rowj = pl.BlockSpec((BK, D), lambda i, j: (j, 0))
    return pl.pallas_call(
        _flash_body,
        grid=(NBQ, NBK),
        in_specs=[rowi, rowj, rowj],
        out_specs=rowi,
        out_shape=jax.ShapeDtypeStruct((MP, D), jnp.float32),
        scratch_shapes=[
            pltpu.VMEM((BQ, D), jnp.float32),
            pltpu.VMEM((BQ, 1), jnp.float32),
            pltpu.VMEM((BQ, 1), jnp.float32),
        ],
        compiler_params=pltpu.CompilerParams(
            dimension_semantics=("parallel", "arbitrary")),
    )(q, k, c)


def _epilogue_body(g_b, z_b, l0_b, l1_b, dn_b, o_b):
    dn = dn_b[...]
    dn = jnp.where(dn == 0.0, 1.0, dn)
    local = (l0_b[...] + l1_b[...]) / dn
    o_b[...] = _leaky(local + g_b[...] + z_b[...])


def _epilogue(glob, zp, l0, l1, dn):
    rowi = pl.BlockSpec((BQ, D), lambda i: (i, 0))
    veci = pl.BlockSpec((BQ, 1), lambda i: (i, 0))
    return pl.pallas_call(
        _epilogue_body,
        grid=(NBQ,),
        in_specs=[rowi, rowi, rowi, rowi, veci],
        out_specs=rowi,
        out_shape=jax.ShapeDtypeStruct((MP, D), jnp.float32),
    )(glob, zp, l0, l1, dn)


def kernel(z, edge_index, Wg, a, Wc, Wq, Wk):
    m, d = z.shape
    assert (m, d, edge_index.shape[1]) == (M, D, E)

    zp = jnp.pad(z, ((0, MP - m), (0, 0)))
    a1 = a[:d].reshape(d, 1).astype(jnp.float32)
    a2 = a[d:].reshape(d, 1).astype(jnp.float32)
    wg, q, k, c, u2, v2 = _projections(
        zp, Wg.T, Wq.T, Wk.T, Wc.T, a1, a2)
    u1 = u2.reshape(MP)
    v1 = v2.reshape(MP)
    vmax16 = jnp.full((16,), jnp.max(v2), jnp.float32)

    src = edge_index[0].astype(jnp.int32)
    dst = edge_index[1].astype(jnp.int32)
    sk = jnp.sort((src << KEY_SHIFT) | dst)
    ssrc = sk >> KEY_SHIFT
    sdst = sk & ((1 << KEY_SHIFT) - 1)
    first = jnp.concatenate(
        [jnp.ones((1,), bool), sk[1:] != sk[:-1]])
    sdstl = jnp.where(first, sdst, M)
    padv = jnp.full((EP - E,), M, jnp.int32)
    ssrc = jnp.concatenate([ssrc, padv])
    sdst = jnp.concatenate([sdst, padv])
    sdstl = jnp.concatenate([sdstl, padv])

    zmat = jnp.zeros((MP, D), jnp.float32)
    zvec = jnp.zeros((MP,), jnp.float32)
    localp, denomp = _sc_local(
        wg, u1, v1, vmax16,
        ssrc.reshape(EP // 128, 128), sdst.reshape(EP // 128, 128),
        sdstl.reshape(EP // 128, 128), zmat, zvec)

    glob = _flash(q.astype(jnp.bfloat16), k.astype(jnp.bfloat16),
                  c.astype(jnp.bfloat16))
    dn = (denomp[0] + denomp[1]).reshape(MP, 1)
    out = _epilogue(glob, zp, localp[0], localp[1], dn)
    return out[:m]


# speedup vs baseline: 2.8913x; 1.1192x over previous
import functools

import jax
import jax.numpy as jnp
from jax import lax
from jax.experimental import pallas as pl
from jax.experimental.pallas import tpu as pltpu
from jax.experimental.pallas import tpu_sc as plsc

M = 10000
D = 128
E = 160000
MP = 10240
EP = 163840
EPT = EP // 32
NCH = EPT // 128
RPT = MP // 16
BQ = 256
BK = 256
NBQ = MP // BQ
NBK = MP // BK
KEY_SHIFT = 14


def _leaky(x):
    return jnp.maximum(x, 0.0) + 0.01 * jnp.minimum(x, 0.0)


def _proj_body(z_b, wgT, wqT, wkT, wcT, a1, a2, wg_o, q_o, k_o, c_o, u_o, v_o,
               qn_o, kn_o):
    zb = z_b[...]
    wg = jnp.dot(zb, wgT[...], preferred_element_type=jnp.float32)
    wg_o[...] = wg
    q = jnp.dot(zb, wqT[...], preferred_element_type=jnp.float32)
    q_o[...] = q
    k = jnp.dot(zb, wkT[...], preferred_element_type=jnp.float32)
    k_o[...] = k
    c_o[...] = jnp.dot(zb, wcT[...], preferred_element_type=jnp.float32)
    u_o[...] = jnp.dot(wg, a1[...], preferred_element_type=jnp.float32)
    v_o[...] = jnp.dot(wg, a2[...], preferred_element_type=jnp.float32)
    qn_o[...] = jnp.sqrt(jnp.sum(q * q, axis=1, keepdims=True))
    kn_o[...] = jnp.sqrt(jnp.sum(k * k, axis=1, keepdims=True))


def _projections(zp, wgT, wqT, wkT, wcT, a1, a2):
    full = pl.BlockSpec((D, D), lambda i: (0, 0))
    colv = pl.BlockSpec((D, 1), lambda i: (0, 0))
    row = pl.BlockSpec((BQ, D), lambda i: (i, 0))
    rowv = pl.BlockSpec((BQ, 1), lambda i: (i, 0))
    mat = jax.ShapeDtypeStruct((MP, D), jnp.float32)
    vec = jax.ShapeDtypeStruct((MP, 1), jnp.float32)
    return pl.pallas_call(
        _proj_body,
        grid=(NBQ,),
        in_specs=[row, full, full, full, full, colv, colv],
        out_specs=[row, row, row, row, rowv, rowv, rowv, rowv],
        out_shape=[mat, mat, mat, mat, vec, vec, vec, vec],
    )(zp, wgT, wqT, wkT, wcT, a1, a2)


def _sc_body(wg_hbm, u_hbm, v_hbm, vmax_hbm, s2d_hbm, d2d_hbm, dl2d_hbm,
             zmat_hbm, zvec_hbm,
             local_out, denom_out,
             vm_v, es_v, s2v, d2v, dl2v, uvals_v, vvals_v, rows_v,
             u_sp, v_sp, local_sp, denom_sp):
    c = lax.axis_index("c")
    s = lax.axis_index("s")
    w = s * 2 + c

    pltpu.sync_copy(zmat_hbm.at[pl.ds(s * RPT, RPT)],
                    local_sp.at[pl.ds(s * RPT, RPT)])
    pltpu.sync_copy(zvec_hbm.at[pl.ds(s * RPT, RPT)],
                    denom_sp.at[pl.ds(s * RPT, RPT)])
    pltpu.sync_copy(u_hbm.at[pl.ds(s * RPT, RPT)],
                    u_sp.at[pl.ds(s * RPT, RPT)])
    pltpu.sync_copy(v_hbm.at[pl.ds(s * RPT, RPT)],
                    v_sp.at[pl.ds(s * RPT, RPT)])

    pltpu.sync_copy(vmax_hbm, vm_v)
    pltpu.sync_copy(s2d_hbm.at[pl.ds(w * NCH, NCH)], s2v)
    pltpu.sync_copy(d2d_hbm.at[pl.ds(w * NCH, NCH)], d2v)
    pltpu.sync_copy(dl2d_hbm.at[pl.ds(w * NCH, NCH)], dl2v)

    vmaxb = vm_v[...]

    plsc.subcore_barrier()

    def p1(j, carry):
        pltpu.sync_copy(u_sp.at[s2v.at[j]], uvals_v)
        pltpu.sync_copy(v_sp.at[d2v.at[j]], vvals_v)
        for g in range(8):
            us = uvals_v[pl.ds(g * 16, 16)]
            vd = vvals_v[pl.ds(g * 16, 16)]
            e = _leaky(us + vd)
            shift = _leaky(us + vmaxb)
            es_v[pl.ds(j * 128 + g * 16, 16)] = jnp.exp(e - shift)
        pltpu.sync_copy(es_v.at[pl.ds(j * 128, 128)],
                        denom_sp.at[s2v.at[j]], add=True)
        return carry
    lax.fori_loop(0, NCH, p1, 0)

    def p2(j, carry):
        pltpu.sync_copy(wg_hbm.at[dl2v.at[j]], rows_v)

        def scale(r, cc):
            idx = jnp.full((16,), j * 128 + r, jnp.int32)
            esb = plsc.load_gather(es_v, [idx])
            for g in range(D // 16):
                rows_v[r, pl.ds(g * 16, 16)] = (
                    rows_v[r, pl.ds(g * 16, 16)] * esb)
            return cc
        lax.fori_loop(0, 128, scale, 0)

        pltpu.sync_copy(rows_v, local_sp.at[s2v.at[j]], add=True)
        return carry
    lax.fori_loop(0, NCH, p2, 0)

    plsc.subcore_barrier()
    pltpu.sync_copy(local_sp.at[pl.ds(s * RPT, RPT)],
                    local_out.at[c].at[pl.ds(s * RPT, RPT)])
    pltpu.sync_copy(denom_sp.at[pl.ds(s * RPT, RPT)],
                    denom_out.at[c].at[pl.ds(s * RPT, RPT)])


def _sc_local(wg, u1, v1, vmax16, s2d, d2d, dl2d, zmat, zvec):
    mesh = plsc.VectorSubcoreMesh(core_axis_name="c", subcore_axis_name="s",
                                  num_cores=2, num_subcores=16)
    return pl.kernel(
        _sc_body,
        out_type=(jax.ShapeDtypeStruct((2, MP, D), jnp.float32),
                  jax.ShapeDtypeStruct((2, MP), jnp.float32)),
        mesh=mesh,
        scratch_types=[
            pltpu.VMEM((16,), jnp.float32),
            pltpu.VMEM((EPT,), jnp.float32),
            pltpu.VMEM((NCH, 128), jnp.int32),
            pltpu.VMEM((NCH, 128), jnp.int32),
            pltpu.VMEM((NCH, 128), jnp.int32),
            pltpu.VMEM((128,), jnp.float32),
            pltpu.VMEM((128,), jnp.float32),
            pltpu.VMEM((128, D), jnp.float32),
            pltpu.VMEM_SHARED((MP,), jnp.float32),
            pltpu.VMEM_SHARED((MP,), jnp.float32),
            pltpu.VMEM_SHARED((MP, D), jnp.float32),
            pltpu.VMEM_SHARED((MP,), jnp.float32),
        ],
        compiler_params=pltpu.CompilerParams(needs_layout_passes=False),
    )(wg, u1, v1, vmax16, s2d, d2d, dl2d, zmat, zvec)


def _flash_body(q_b, k_b, c_b, bnd_b, o_b, acc, lv):
    j = pl.program_id(1)

    @pl.when(j == 0)
    def _():
        acc[...] = jnp.zeros_like(acc)
        lv[...] = jnp.zeros_like(lv)

    s = lax.dot_general(q_b[...], k_b[...], (((1,), (1,)), ((), ())),
                        preferred_element_type=jnp.float32)
    p = jnp.exp(s - bnd_b[...])
    lv[...] = lv[...] + jnp.sum(p, axis=1, keepdims=True)
    acc[...] = acc[...] + jnp.dot(p.astype(jnp.bfloat16), c_b[...],
                                  preferred_element_type=jnp.float32)

    @pl.when(j == NBK - 1)
    def _():
        pad = jnp.exp(-bnd_b[...]) * float(MP - M)
        o_b[...] = acc[...] / (lv[...] - pad)


def _flash(q, k, c, bound):
    rowi = pl.BlockSpec((BQ, D), lambda i, j: (i, 0))
    rowj = pl.BlockSpec((BK, D), lambda i, j: (j, 0))
    veci = pl.BlockSpec((BQ, 1), lambda i, j: (i, 0))
    return pl.pallas_call(
        _flash_body,
        grid=(NBQ, NBK),
        in_specs=[rowi, rowj, rowj, veci],
        out_specs=rowi,
        out_shape=jax.ShapeDtypeStruct((MP, D), jnp.float32),
        scratch_shapes=[
            pltpu.VMEM((BQ, D), jnp.float32),
            pltpu.VMEM((BQ, 1), jnp.float32),
        ],
        compiler_params=pltpu.CompilerParams(
            dimension_semantics=("parallel", "arbitrary")),
    )(q, k, c, bound)


def _epilogue_body(g_b, z_b, l0_b, l1_b, dn_b, o_b):
    dn = dn_b[...]
    dn = jnp.where(dn == 0.0, 1.0, dn)
    local = (l0_b[...] + l1_b[...]) / dn
    o_b[...] = _leaky(local + g_b[...] + z_b[...])


def _epilogue(glob, zp, l0, l1, dn):
    rowi = pl.BlockSpec((BQ, D), lambda i: (i, 0))
    veci = pl.BlockSpec((BQ, 1), lambda i: (i, 0))
    return pl.pallas_call(
        _epilogue_body,
        grid=(NBQ,),
        in_specs=[rowi, rowi, rowi, rowi, veci],
        out_specs=rowi,
        out_shape=jax.ShapeDtypeStruct((MP, D), jnp.float32),
    )(glob, zp, l0, l1, dn)


def kernel(z, edge_index, Wg, a, Wc, Wq, Wk):
    m, d = z.shape
    assert (m, d, edge_index.shape[1]) == (M, D, E)

    zp = jnp.pad(z, ((0, MP - m), (0, 0)))
    a1 = a[:d].reshape(d, 1).astype(jnp.float32)
    a2 = a[d:].reshape(d, 1).astype(jnp.float32)
    wg, q, k, c, u2, v2, qn, kn = _projections(
        zp, Wg.T, Wq.T, Wk.T, Wc.T, a1, a2)
    u1 = u2.reshape(MP)
    v1 = v2.reshape(MP)
    vmax16 = jnp.full((16,), jnp.max(v2), jnp.float32)

    src = edge_index[0].astype(jnp.int32)
    dst = edge_index[1].astype(jnp.int32)
    sk = jnp.sort((src << KEY_SHIFT) | dst)
    ssrc = sk >> KEY_SHIFT
    sdst = sk & ((1 << KEY_SHIFT) - 1)
    first = jnp.concatenate(
        [jnp.ones((1,), bool), sk[1:] != sk[:-1]])
    sdstl = jnp.where(first, sdst, M)
    padv = jnp.full((EP - E,), M, jnp.int32)
    ssrc = jnp.concatenate([ssrc, padv])
    sdst = jnp.concatenate([sdst, padv])
    sdstl = jnp.concatenate([sdstl, padv])

    zmat = jnp.zeros((MP, D), jnp.float32)
    zvec = jnp.zeros((MP,), jnp.float32)
    localp, denomp = _sc_local(
        wg, u1, v1, vmax16,
        ssrc.reshape(EP // 128, 128), sdst.reshape(EP // 128, 128),
        sdstl.reshape(EP // 128, 128), zmat, zvec)

    scale = 1.0 / (float(D) ** 0.5)
    bound = qn * (jnp.max(kn) * scale)
    glob = _flash((q * scale).astype(jnp.bfloat16), k.astype(jnp.bfloat16),
                  c.astype(jnp.bfloat16), bound)
    dn = (denomp[0] + denomp[1]).reshape(MP, 1)
    out = _epilogue(glob, zp, localp[0], localp[1], dn)
    return out[:m]


# speedup vs baseline: 2.9225x; 1.0108x over previous
import functools

import jax
import jax.numpy as jnp
from jax import lax
from jax.experimental import pallas as pl
from jax.experimental.pallas import tpu as pltpu
from jax.experimental.pallas import tpu_sc as plsc

M = 10000
D = 128
E = 160000
MP = 10240
EP = 163840
EPT = EP // 32
NCH = EPT // 128
NR = 10016
RPT = 640
RLAST = NR - 15 * RPT
BQ = 256
BK = 256
NBQ = MP // BQ
NBK = MP // BK
KEY_SHIFT = 14


def _leaky(x):
    return jnp.maximum(x, 0.0) + 0.01 * jnp.minimum(x, 0.0)


def _proj_body(z_b, wgT, wqT, wkT, wcT, a1, a2, wg_o, q_o, k_o, c_o, u_o, v_o,
               qn_o, kn_o):
    zb = z_b[...]
    wg = jnp.dot(zb, wgT[...], preferred_element_type=jnp.float32)
    wg_o[...] = wg
    q = jnp.dot(zb, wqT[...], preferred_element_type=jnp.float32)
    q_o[...] = q
    k = jnp.dot(zb, wkT[...], preferred_element_type=jnp.float32)
    k_o[...] = k
    c_o[...] = jnp.dot(zb, wcT[...], preferred_element_type=jnp.float32)
    u_o[...] = jnp.dot(wg, a1[...], preferred_element_type=jnp.float32)
    v_o[...] = jnp.dot(wg, a2[...], preferred_element_type=jnp.float32)
    qn_o[...] = jnp.sqrt(jnp.sum(q * q, axis=1, keepdims=True))
    kn_o[...] = jnp.sqrt(jnp.sum(k * k, axis=1, keepdims=True))


def _projections(zp, wgT, wqT, wkT, wcT, a1, a2):
    full = pl.BlockSpec((D, D), lambda i: (0, 0))
    colv = pl.BlockSpec((D, 1), lambda i: (0, 0))
    row = pl.BlockSpec((BQ, D), lambda i: (i, 0))
    rowv = pl.BlockSpec((BQ, 1), lambda i: (i, 0))
    mat = jax.ShapeDtypeStruct((MP, D), jnp.float32)
    vec = jax.ShapeDtypeStruct((MP, 1), jnp.float32)
    return pl.pallas_call(
        _proj_body,
        grid=(NBQ,),
        in_specs=[row, full, full, full, full, colv, colv],
        out_specs=[row, row, row, row, rowv, rowv, rowv, rowv],
        out_shape=[mat, mat, mat, mat, vec, vec, vec, vec],
    )(zp, wgT, wqT, wkT, wcT, a1, a2)


def _sc_body(wg_hbm, u_hbm, v_hbm, vmax_hbm, s2d_hbm, d2d_hbm, dl2d_hbm,
             zmat_hbm, zvec_hbm,
             local_out, denom_out,
             vm_v, es0, es1, s2v, d2v, dl2v, uvals_v, vvals_v,
             rows0, rows1, gs0, gs1, ss0, ss1,
             u_sp, v_sp, local_sp, denom_sp):
    esb = (es0, es1)
    rows = (rows0, rows1)
    gsem = (gs0, gs1)
    ssem = (ss0, ss1)
    c = lax.axis_index("c")
    s = lax.axis_index("s")
    w = s * 2 + c

    @pl.when(s < 15)
    def _():
        pltpu.sync_copy(zmat_hbm.at[pl.ds(s * RPT, RPT)],
                        local_sp.at[pl.ds(s * RPT, RPT)])

    @pl.when(s == 15)
    def _():
        pltpu.sync_copy(zmat_hbm.at[pl.ds(15 * RPT, RLAST)],
                        local_sp.at[pl.ds(15 * RPT, RLAST)])

    pltpu.sync_copy(zvec_hbm.at[pl.ds(s * RPT, RPT)],
                    denom_sp.at[pl.ds(s * RPT, RPT)])

    pltpu.sync_copy(u_hbm.at[pl.ds(s * RPT, RPT)],
                    u_sp.at[pl.ds(s * RPT, RPT)])
    pltpu.sync_copy(v_hbm.at[pl.ds(s * RPT, RPT)],
                    v_sp.at[pl.ds(s * RPT, RPT)])

    pltpu.sync_copy(vmax_hbm, vm_v)
    pltpu.sync_copy(s2d_hbm.at[pl.ds(w * NCH, NCH)], s2v)
    pltpu.sync_copy(d2d_hbm.at[pl.ds(w * NCH, NCH)], d2v)
    pltpu.sync_copy(dl2d_hbm.at[pl.ds(w * NCH, NCH)], dl2v)

    vmaxb = vm_v[...]

    plsc.subcore_barrier()

    def compute_es(jj, dst):
        pltpu.sync_copy(u_sp.at[s2v.at[jj]], uvals_v)
        pltpu.sync_copy(v_sp.at[d2v.at[jj]], vvals_v)
        for g in range(8):
            us = uvals_v[pl.ds(g * 16, 16)]
            vd = vvals_v[pl.ds(g * 16, 16)]
            e = _leaky(us + vd)
            shift = _leaky(us + vmaxb)
            dst[pl.ds(g * 16, 16)] = jnp.exp(e - shift)
        pltpu.sync_copy(dst, denom_sp.at[s2v.at[jj]], add=True)

    pltpu.async_copy(wg_hbm.at[dl2v.at[0]], rows[0], gsem[0])
    compute_es(0, esb[0])

    def pair(g2, carry):
        for bb in range(2):
            j = g2 * 2 + bb
            ob = 1 - bb

            @pl.when(j >= 1)
            def _():
                pltpu.make_async_copy(
                    rows[ob], local_sp.at[s2v.at[j - 1]], ssem[ob]).wait()

            @pl.when(j + 1 < NCH)
            def _():
                pltpu.async_copy(wg_hbm.at[dl2v.at[j + 1]], rows[ob],
                                 gsem[ob])
                compute_es(j + 1, esb[ob])

            pltpu.make_async_copy(
                wg_hbm.at[dl2v.at[j]], rows[bb], gsem[bb]).wait()

            def scale(r, cc):
                idx = jnp.full((16,), r, jnp.int32)
                ev = plsc.load_gather(esb[bb], [idx])
                for gg in range(D // 16):
                    rows[bb][r, pl.ds(gg * 16, 16)] = (
                        rows[bb][r, pl.ds(gg * 16, 16)] * ev)
                return cc
            lax.fori_loop(0, 128, scale, 0)

            pltpu.async_copy(rows[bb], local_sp.at[s2v.at[j]], ssem[bb],
                             add=True)
        return carry
    lax.fori_loop(0, NCH // 2, pair, 0)

    pltpu.make_async_copy(
        rows[1], local_sp.at[s2v.at[NCH - 1]], ssem[1]).wait()

    plsc.subcore_barrier()

    @pl.when(s < 15)
    def _():
        pltpu.sync_copy(local_sp.at[pl.ds(s * RPT, RPT)],
                        local_out.at[c].at[pl.ds(s * RPT, RPT)])

    @pl.when(s == 15)
    def _():
        pltpu.sync_copy(local_sp.at[pl.ds(15 * RPT, RLAST)],
                        local_out.at[c].at[pl.ds(15 * RPT, RLAST)])

    pltpu.sync_copy(denom_sp.at[pl.ds(s * RPT, RPT)],
                    denom_out.at[c].at[pl.ds(s * RPT, RPT)])


def _sc_local(wg, u1, v1, vmax16, s2d, d2d, dl2d, zmat, zvec):
    mesh = plsc.VectorSubcoreMesh(core_axis_name="c", subcore_axis_name="s",
                                  num_cores=2, num_subcores=16)
    return pl.kernel(
        _sc_body,
        out_type=(jax.ShapeDtypeStruct((2, MP, D), jnp.float32),
                  jax.ShapeDtypeStruct((2, MP), jnp.float32)),
        mesh=mesh,
        scratch_types=[
            pltpu.VMEM((16,), jnp.float32),
            pltpu.VMEM((128,), jnp.float32),
            pltpu.VMEM((128,), jnp.float32),
            pltpu.VMEM((NCH, 128), jnp.int32),
            pltpu.VMEM((NCH, 128), jnp.int32),
            pltpu.VMEM((NCH, 128), jnp.int32),
            pltpu.VMEM((128,), jnp.float32),
            pltpu.VMEM((128,), jnp.float32),
            pltpu.VMEM((128, D), jnp.float32),
            pltpu.VMEM((128, D), jnp.float32),
            pltpu.SemaphoreType.DMA,
            pltpu.SemaphoreType.DMA,
            pltpu.SemaphoreType.DMA,
            pltpu.SemaphoreType.DMA,
            pltpu.VMEM_SHARED((MP,), jnp.float32),
            pltpu.VMEM_SHARED((MP,), jnp.float32),
            pltpu.VMEM_SHARED((NR, D), jnp.float32),
            pltpu.VMEM_SHARED((MP,), jnp.float32),
        ],
        compiler_params=pltpu.CompilerParams(needs_layout_passes=False),
        cost_estimate=pl.CostEstimate(flops=4 * EP * D,
                                      bytes_accessed=2 * EP * D * 4,
                                      transcendentals=EP),
    )(wg, u1, v1, vmax16, s2d, d2d, dl2d, zmat, zvec)


def _flash_body(q_b, k_b, c_b, bnd_b, o_b, acc, lv):
    j = pl.program_id(1)

    @pl.when(j == 0)
    def _():
        acc[...] = jnp.zeros_like(acc)
        lv[...] = jnp.zeros_like(lv)

    s = lax.dot_general(q_b[...], k_b[...], (((1,), (1,)), ((), ())),
                        preferred_element_type=jnp.float32)
    p = jnp.exp(s - bnd_b[...])
    lv[...] = lv[...] + jnp.sum(p, axis=1, keepdims=True)
    acc[...] = acc[...] + jnp.dot(p.astype(jnp.bfloat16), c_b[...],
                                  preferred_element_type=jnp.float32)

    @pl.when(j == NBK - 1)
    def _():
        pad = jnp.exp(-bnd_b[...]) * float(MP - M)
        o_b[...] = acc[...] / (lv[...] - pad)


def _flash(q, k, c, bound):
    rowi = pl.BlockSpec((BQ, D), lambda i, j: (i, 0))
    rowj = pl.BlockSpec((BK, D), lambda i, j: (j, 0))
    veci = pl.BlockSpec((BQ, 1), lambda i, j: (i, 0))
    return pl.pallas_call(
        _flash_body,
        grid=(NBQ, NBK),
        in_specs=[rowi, rowj, rowj, veci],
        out_specs=rowi,
        out_shape=jax.ShapeDtypeStruct((MP, D), jnp.float32),
        scratch_shapes=[
            pltpu.VMEM((BQ, D), jnp.float32),
            pltpu.VMEM((BQ, 1), jnp.float32),
        ],
        compiler_params=pltpu.CompilerParams(
            dimension_semantics=("parallel", "arbitrary")),
    )(q, k, c, bound)


def _epilogue_body(g_b, z_b, l0_b, l1_b, dn_b, o_b):
    dn = dn_b[...]
    dn = jnp.where(dn == 0.0, 1.0, dn)
    local = (l0_b[...] + l1_b[...]) / dn
    o_b[...] = _leaky(local + g_b[...] + z_b[...])


def _epilogue(glob, zp, l0, l1, dn):
    rowi = pl.BlockSpec((BQ, D), lambda i: (i, 0))
    veci = pl.BlockSpec((BQ, 1), lambda i: (i, 0))
    return pl.pallas_call(
        _epilogue_body,
        grid=(NBQ,),
        in_specs=[rowi, rowi, rowi, rowi, veci],
        out_specs=rowi,
        out_shape=jax.ShapeDtypeStruct((MP, D), jnp.float32),
    )(glob, zp, l0, l1, dn)


def kernel(z, edge_index, Wg, a, Wc, Wq, Wk):
    m, d = z.shape
    assert (m, d, edge_index.shape[1]) == (M, D, E)

    zp = jnp.pad(z, ((0, MP - m), (0, 0)))
    a1 = a[:d].reshape(d, 1).astype(jnp.float32)
    a2 = a[d:].reshape(d, 1).astype(jnp.float32)
    wg, q, k, c, u2, v2, qn, kn = _projections(
        zp, Wg.T, Wq.T, Wk.T, Wc.T, a1, a2)
    u1 = u2.reshape(MP)
    v1 = v2.reshape(MP)
    vmax16 = jnp.full((16,), jnp.max(v2), jnp.float32)

    src = edge_index[0].astype(jnp.int32)
    dst = edge_index[1].astype(jnp.int32)
    sk = jnp.sort((src << KEY_SHIFT) | dst)
    ssrc = sk >> KEY_SHIFT
    sdst = sk & ((1 << KEY_SHIFT) - 1)
    first = jnp.concatenate(
        [jnp.ones((1,), bool), sk[1:] != sk[:-1]])
    sdstl = jnp.where(first, sdst, M)
    padv = jnp.full((EP - E,), M, jnp.int32)
    ssrc = jnp.concatenate([ssrc, padv])
    sdst = jnp.concatenate([sdst, padv])
    sdstl = jnp.concatenate([sdstl, padv])

    zmat = jnp.zeros((NR, D), jnp.float32)
    zvec = jnp.zeros((MP,), jnp.float32)
    localp, denomp = _sc_local(
        wg, u1, v1, vmax16,
        ssrc.reshape(EP // 128, 128), sdst.reshape(EP // 128, 128),
        sdstl.reshape(EP // 128, 128), zmat, zvec)

    scale = 1.0 / (float(D) ** 0.5)
    bound = qn * (jnp.max(kn) * scale)
    glob = _flash((q * scale).astype(jnp.bfloat16), k.astype(jnp.bfloat16),
                  c.astype(jnp.bfloat16), bound)
    dn = (denomp[0] + denomp[1]).reshape(MP, 1)
    out = _epilogue(glob, zp, localp[0], localp[1], dn)
    return out[:m]


# speedup vs baseline: 2.9270x; 1.0016x over previous
import functools

import jax
import jax.numpy as jnp
from jax import lax
from jax.experimental import pallas as pl
from jax.experimental.pallas import tpu as pltpu
from jax.experimental.pallas import tpu_sc as plsc

M = 10000
D = 128
E = 160000
MP = 10240
EP = 163840
EPT = EP // 32
NCH = EPT // 128
NR = 10016
RPT = 640
RLAST = NR - 15 * RPT
BQ = 256
BK = 256
NBQ = MP // BQ
NBK = MP // BK
KEY_SHIFT = 14


def _leaky(x):
    return jnp.maximum(x, 0.0) + 0.01 * jnp.minimum(x, 0.0)


def _proj_body(z_b, wgT, wqT, wkT, wcT, a1, a2, wg_o, q_o, k_o, c_o, u_o, v_o,
               qn_o, kn_o):
    zb = z_b[...]
    wg = jnp.dot(zb, wgT[...], preferred_element_type=jnp.float32)
    wg_o[...] = wg
    q = jnp.dot(zb, wqT[...], preferred_element_type=jnp.float32)
    q_o[...] = q
    k = jnp.dot(zb, wkT[...], preferred_element_type=jnp.float32)
    k_o[...] = k
    c_o[...] = jnp.dot(zb, wcT[...], preferred_element_type=jnp.float32)
    u_o[...] = jnp.dot(wg, a1[...], preferred_element_type=jnp.float32)
    v_o[...] = jnp.dot(wg, a2[...], preferred_element_type=jnp.float32)
    qn_o[...] = jnp.sqrt(jnp.sum(q * q, axis=1, keepdims=True))
    kn_o[...] = jnp.sqrt(jnp.sum(k * k, axis=1, keepdims=True))


def _projections(zp, wgT, wqT, wkT, wcT, a1, a2):
    full = pl.BlockSpec((D, D), lambda i: (0, 0))
    colv = pl.BlockSpec((D, 1), lambda i: (0, 0))
    row = pl.BlockSpec((BQ, D), lambda i: (i, 0))
    rowv = pl.BlockSpec((BQ, 1), lambda i: (i, 0))
    mat = jax.ShapeDtypeStruct((MP, D), jnp.float32)
    vec = jax.ShapeDtypeStruct((MP, 1), jnp.float32)
    return pl.pallas_call(
        _proj_body,
        grid=(NBQ,),
        in_specs=[row, full, full, full, full, colv, colv],
        out_specs=[row, row, row, row, rowv, rowv, rowv, rowv],
        out_shape=[mat, mat, mat, mat, vec, vec, vec, vec],
    )(zp, wgT, wqT, wkT, wcT, a1, a2)


def _sc_body(wg_hbm, u_hbm, v_hbm, vmax_hbm, s2d_hbm, d2d_hbm, dl2d_hbm,
             zmat_hbm, zvec_hbm,
             local_out, denom_out,
             vm_v, es0, es1, s2v, d2v, dl2v, uvals_v, vvals_v,
             rows0, rows1, gs0, gs1, ss0, ss1,
             u_sp, v_sp, local_sp, denom_sp):
    esb = (es0, es1)
    rows = (rows0, rows1)
    gsem = (gs0, gs1)
    ssem = (ss0, ss1)
    c = lax.axis_index("c")
    s = lax.axis_index("s")
    w = s * 2 + c

    @pl.when(s < 15)
    def _():
        pltpu.sync_copy(zmat_hbm.at[pl.ds(s * RPT, RPT)],
                        local_sp.at[pl.ds(s * RPT, RPT)])

    @pl.when(s == 15)
    def _():
        pltpu.sync_copy(zmat_hbm.at[pl.ds(15 * RPT, RLAST)],
                        local_sp.at[pl.ds(15 * RPT, RLAST)])

    pltpu.sync_copy(zvec_hbm.at[pl.ds(s * RPT, RPT)],
                    denom_sp.at[pl.ds(s * RPT, RPT)])

    pltpu.sync_copy(u_hbm.at[pl.ds(s * RPT, RPT)],
                    u_sp.at[pl.ds(s * RPT, RPT)])
    pltpu.sync_copy(v_hbm.at[pl.ds(s * RPT, RPT)],
                    v_sp.at[pl.ds(s * RPT, RPT)])

    pltpu.sync_copy(vmax_hbm, vm_v)
    pltpu.sync_copy(s2d_hbm.at[pl.ds(w * NCH, NCH)], s2v)
    pltpu.sync_copy(d2d_hbm.at[pl.ds(w * NCH, NCH)], d2v)
    pltpu.sync_copy(dl2d_hbm.at[pl.ds(w * NCH, NCH)], dl2v)

    vmaxb = vm_v[...]

    plsc.subcore_barrier()

    def compute_es(jj, dst):
        pltpu.sync_copy(u_sp.at[s2v.at[jj]], uvals_v)
        pltpu.sync_copy(v_sp.at[d2v.at[jj]], vvals_v)
        for g in range(8):
            us = uvals_v[pl.ds(g * 16, 16)]
            vd = vvals_v[pl.ds(g * 16, 16)]
            e = _leaky(us + vd)
            shift = _leaky(us + vmaxb)
            dst[pl.ds(g * 16, 16)] = jnp.exp(e - shift)
        pltpu.sync_copy(dst, denom_sp.at[s2v.at[jj]], add=True)

    pltpu.async_copy(wg_hbm.at[dl2v.at[0]], rows[0], gsem[0])
    compute_es(0, esb[0])

    def pair(g2, carry):
        for bb in range(2):
            j = g2 * 2 + bb
            ob = 1 - bb

            @pl.when(j >= 1)
            def _():
                pltpu.make_async_copy(
                    rows[ob], local_sp.at[s2v.at[j - 1]], ssem[ob]).wait()

            @pl.when(j + 1 < NCH)
            def _():
                pltpu.async_copy(wg_hbm.at[dl2v.at[j + 1]], rows[ob],
                                 gsem[ob])
                compute_es(j + 1, esb[ob])

            pltpu.make_async_copy(
                wg_hbm.at[dl2v.at[j]], rows[bb], gsem[bb]).wait()

            def scale(rg, cc):
                evg = esb[bb][pl.ds(rg * 16, 16)]
                for rr in range(16):
                    ev = lax.gather(
                        evg, jnp.full((16, 1), rr, jnp.int32),
                        lax.GatherDimensionNumbers(
                            offset_dims=(), collapsed_slice_dims=(0,),
                            start_index_map=(0,)),
                        (1,), mode=lax.GatherScatterMode.PROMISE_IN_BOUNDS)
                    r = rg * 16 + rr
                    for gg in range(D // 16):
                        rows[bb][r, pl.ds(gg * 16, 16)] = (
                            rows[bb][r, pl.ds(gg * 16, 16)] * ev)
                return cc
            lax.fori_loop(0, 8, scale, 0)

            pltpu.async_copy(rows[bb], local_sp.at[s2v.at[j]], ssem[bb],
                             add=True)
        return carry
    lax.fori_loop(0, NCH // 2, pair, 0)

    pltpu.make_async_copy(
        rows[1], local_sp.at[s2v.at[NCH - 1]], ssem[1]).wait()

    plsc.subcore_barrier()

    @pl.when(s < 15)
    def _():
        pltpu.sync_copy(local_sp.at[pl.ds(s * RPT, RPT)],
                        local_out.at[c].at[pl.ds(s * RPT, RPT)])

    @pl.when(s == 15)
    def _():
        pltpu.sync_copy(local_sp.at[pl.ds(15 * RPT, RLAST)],
                        local_out.at[c].at[pl.ds(15 * RPT, RLAST)])

    pltpu.sync_copy(denom_sp.at[pl.ds(s * RPT, RPT)],
                    denom_out.at[c].at[pl.ds(s * RPT, RPT)])


def _sc_local(wg, u1, v1, vmax16, s2d, d2d, dl2d, zmat, zvec):
    mesh = plsc.VectorSubcoreMesh(core_axis_name="c", subcore_axis_name="s",
                                  num_cores=2, num_subcores=16)
    return pl.kernel(
        _sc_body,
        out_type=(jax.ShapeDtypeStruct((2, MP, D), jnp.float32),
                  jax.ShapeDtypeStruct((2, MP), jnp.float32)),
        mesh=mesh,
        scratch_types=[
            pltpu.VMEM((16,), jnp.float32),
            pltpu.VMEM((128,), jnp.float32),
            pltpu.VMEM((128,), jnp.float32),
            pltpu.VMEM((NCH, 128), jnp.int32),
            pltpu.VMEM((NCH, 128), jnp.int32),
            pltpu.VMEM((NCH, 128), jnp.int32),
            pltpu.VMEM((128,), jnp.float32),
            pltpu.VMEM((128,), jnp.float32),
            pltpu.VMEM((128, D), jnp.float32),
            pltpu.VMEM((128, D), jnp.float32),
            pltpu.SemaphoreType.DMA,
            pltpu.SemaphoreType.DMA,
            pltpu.SemaphoreType.DMA,
            pltpu.SemaphoreType.DMA,
            pltpu.VMEM_SHARED((MP,), jnp.float32),
            pltpu.VMEM_SHARED((MP,), jnp.float32),
            pltpu.VMEM_SHARED((NR, D), jnp.float32),
            pltpu.VMEM_SHARED((MP,), jnp.float32),
        ],
        compiler_params=pltpu.CompilerParams(needs_layout_passes=False),
        cost_estimate=pl.CostEstimate(flops=4 * EP * D,
                                      bytes_accessed=2 * EP * D * 4,
                                      transcendentals=EP),
    )(wg, u1, v1, vmax16, s2d, d2d, dl2d, zmat, zvec)


def _flash_body(q_b, k_b, c_b, bnd_b, o_b, acc, lv):
    j = pl.program_id(1)

    @pl.when(j == 0)
    def _():
        acc[...] = jnp.zeros_like(acc)
        lv[...] = jnp.zeros_like(lv)

    s = lax.dot_general(q_b[...], k_b[...], (((1,), (1,)), ((), ())),
                        preferred_element_type=jnp.float32)
    p = jnp.exp(s - bnd_b[...])
    lv[...] = lv[...] + jnp.sum(p, axis=1, keepdims=True)
    acc[...] = acc[...] + jnp.dot(p.astype(jnp.bfloat16), c_b[...],
                                  preferred_element_type=jnp.float32)

    @pl.when(j == NBK - 1)
    def _():
        pad = jnp.exp(-bnd_b[...]) * float(MP - M)
        o_b[...] = acc[...] / (lv[...] - pad)


def _flash(q, k, c, bound):
    rowi = pl.BlockSpec((BQ, D), lambda i, j: (i, 0))
    rowj = pl.BlockSpec((BK, D), lambda i, j: (j, 0))
    veci = pl.BlockSpec((BQ, 1), lambda i, j: (i, 0))
    return pl.pallas_call(
        _flash_body,
        grid=(NBQ, NBK),
        in_specs=[rowi, rowj, rowj, veci],
        out_specs=rowi,
        out_shape=jax.ShapeDtypeStruct((MP, D), jnp.float32),
        scratch_shapes=[
            pltpu.VMEM((BQ, D), jnp.float32),
            pltpu.VMEM((BQ, 1), jnp.float32),
        ],
        compiler_params=pltpu.CompilerParams(
            dimension_semantics=("parallel", "arbitrary")),
    )(q, k, c, bound)


def _epilogue_body(g_b, z_b, l0_b, l1_b, dn_b, o_b):
    dn = dn_b[...]
    dn = jnp.where(dn == 0.0, 1.0, dn)
    local = (l0_b[...] + l1_b[...]) / dn
    o_b[...] = _leaky(local + g_b[...] + z_b[...])


def _epilogue(glob, zp, l0, l1, dn):
    rowi = pl.BlockSpec((BQ, D), lambda i: (i, 0))
    veci = pl.BlockSpec((BQ, 1), lambda i: (i, 0))
    return pl.pallas_call(
        _epilogue_body,
        grid=(NBQ,),
        in_specs=[rowi, rowi, rowi, rowi, veci],
        out_specs=rowi,
        out_shape=jax.ShapeDtypeStruct((MP, D), jnp.float32),
    )(glob, zp, l0, l1, dn)


def kernel(z, edge_index, Wg, a, Wc, Wq, Wk):
    m, d = z.shape
    assert (m, d, edge_index.shape[1]) == (M, D, E)

    zp = jnp.pad(z, ((0, MP - m), (0, 0)))
    a1 = a[:d].reshape(d, 1).astype(jnp.float32)
    a2 = a[d:].reshape(d, 1).astype(jnp.float32)
    wg, q, k, c, u2, v2, qn, kn = _projections(
        zp, Wg.T, Wq.T, Wk.T, Wc.T, a1, a2)
    u1 = u2.reshape(MP)
    v1 = v2.reshape(MP)
    vmax16 = jnp.full((16,), jnp.max(v2), jnp.float32)

    src = edge_index[0].astype(jnp.int32)
    dst = edge_index[1].astype(jnp.int32)
    sk = jnp.sort((src << KEY_SHIFT) | dst)
    ssrc = sk >> KEY_SHIFT
    sdst = sk & ((1 << KEY_SHIFT) - 1)
    first = jnp.concatenate(
        [jnp.ones((1,), bool), sk[1:] != sk[:-1]])
    sdstl = jnp.where(first, sdst, M)
    padv = jnp.full((EP - E,), M, jnp.int32)
    ssrc = jnp.concatenate([ssrc, padv])
    sdst = jnp.concatenate([sdst, padv])
    sdstl = jnp.concatenate([sdstl, padv])

    zmat = jnp.zeros((NR, D), jnp.float32)
    zvec = jnp.zeros((MP,), jnp.float32)
    localp, denomp = _sc_local(
        wg, u1, v1, vmax16,
        ssrc.reshape(EP // 128, 128), sdst.reshape(EP // 128, 128),
        sdstl.reshape(EP // 128, 128), zmat, zvec)

    scale = 1.0 / (float(D) ** 0.5)
    bound = qn * (jnp.max(kn) * scale)
    glob = _flash((q * scale).astype(jnp.bfloat16), k.astype(jnp.bfloat16),
                  c.astype(jnp.bfloat16), bound)
    dn = (denomp[0] + denomp[1]).reshape(MP, 1)
    out = _epilogue(glob, zp, localp[0], localp[1], dn)
    return out[:m]
